# Initial kernel scaffold; baseline (speedup 1.0000x reference)
import jax
import jax.numpy as jnp
from jax.experimental import pallas as pl


def kernel(x, edge_index, W1, b1, gamma, beta, W2, b2):
    raise NotImplementedError("write your pallas kernel here")


# speedup vs baseline: 13.5497x; 13.5497x over previous
import functools

import jax
import jax.numpy as jnp
from jax import lax
from jax.experimental import pallas as pl
from jax.experimental.pallas import tpu as pltpu
from jax.experimental.pallas import tpu_sc as plsc

N = 10000
F = 128
H = 128
C = 40
CP = 48
E = 320000
NC = 2
NS = 16
NW = NC * NS
NP = 10240
EW = E // NW
K = 80
NCHUNK = EW // K
RT = NP // NS
BN_EPS = 1e-5
B = 1024
G = NP // B


def _sc_mesh():
    return plsc.VectorSubcoreMesh(core_axis_name="c", subcore_axis_name="s")


def _zero_acc(zbuf, acc, s, d):
    for r in range(16):
        for j in range(d // 16):
            zbuf[r, pl.ds(j * 16, 16)] = jnp.zeros((16,), jnp.float32)
    row0 = s * RT

    def body(i, carry):
        pltpu.sync_copy(zbuf, acc.at[pl.ds(row0 + i * 16, 16)])
        return carry

    lax.fori_loop(0, RT // 16, body, 0)


@functools.cache
def _make_deg():
    @functools.partial(
        pl.kernel,
        out_type=jax.ShapeDtypeStruct((NC * NP, 16), jnp.float32),
        mesh=_sc_mesh(),
        scratch_types=[
            pltpu.VMEM((16, 16), jnp.float32),
            pltpu.VMEM((K, 16), jnp.float32),
            pltpu.VMEM((K,), jnp.int32),
            pltpu.VMEM_SHARED((NP, 16), jnp.float32),
        ],
    )
    def deg_kernel(dst_hbm, out_hbm, zbuf, ones, idx, acc):
        c = lax.axis_index("c")
        s = lax.axis_index("s")
        wid = s * NC + c
        _zero_acc(zbuf, acc, s, 16)
        for r in range(K):
            ones[r, :] = jnp.ones((16,), jnp.float32)
        plsc.subcore_barrier()
        base = wid * EW

        def chunk(k, carry):
            pltpu.sync_copy(dst_hbm.at[pl.ds(base + k * K, K)], idx)
            pltpu.sync_copy(ones, acc.at[idx], add=True)
            return carry

        lax.fori_loop(0, NCHUNK, chunk, 0)
        plsc.subcore_barrier()
        row0 = s * RT
        pltpu.sync_copy(acc.at[pl.ds(row0, RT)], out_hbm.at[pl.ds(c * NP + row0, RT)])

    return deg_kernel


@functools.cache
def _make_prop(d):
    @functools.partial(
        pl.kernel,
        out_type=jax.ShapeDtypeStruct((NC * NP, d), jnp.float32),
        mesh=_sc_mesh(),
        scratch_types=[
            pltpu.VMEM((16, d), jnp.float32),
            pltpu.VMEM((K,), jnp.int32),
            pltpu.VMEM((K,), jnp.int32),
            pltpu.VMEM((K, d), jnp.float32),
            pltpu.VMEM_SHARED((NP, d), jnp.float32),
            pltpu.SemaphoreType.DMA,
        ],
        compiler_params=pltpu.CompilerParams(use_tc_tiling_on_sc=(d % 128 == 0)),
    )
    def prop(y_hbm, src_hbm, dst_hbm, out_hbm, zbuf, idx_s, idx_d, rows, acc, sem):
        c = lax.axis_index("c")
        s = lax.axis_index("s")
        wid = s * NC + c
        _zero_acc(zbuf, acc, s, d)
        plsc.subcore_barrier()
        base = wid * EW

        def chunk(k, carry):
            off = base + k * K
            pltpu.sync_copy(src_hbm.at[pl.ds(off, K)], idx_s)
            pltpu.async_copy(y_hbm.at[idx_s], rows, sem).wait()
            pltpu.sync_copy(dst_hbm.at[pl.ds(off, K)], idx_d)
            pltpu.sync_copy(rows, acc.at[idx_d], add=True)
            return carry

        lax.fori_loop(0, NCHUNK, chunk, 0)
        plsc.subcore_barrier()
        row0 = s * RT
        pltpu.sync_copy(acc.at[pl.ds(row0, RT)], out_hbm.at[pl.ds(c * NP + row0, RT)])

    return prop


def _dis(deg_ref):
    deg = deg_ref[0][:, 0:1] + deg_ref[1][:, 0:1] + 1.0
    return lax.rsqrt(deg)


def _tc1_body(deg_ref, x_ref, w_ref, y_ref):
    y_ref[...] = (
        jnp.dot(x_ref[...], w_ref[...], preferred_element_type=jnp.float32)
        * _dis(deg_ref)
    )


def _tc2a_body(acc_ref, y1_ref, deg_ref, b1_ref, hpre_ref, stats_ref):
    i = pl.program_id(0)
    hp = (acc_ref[0] + acc_ref[1] + y1_ref[...]) * _dis(deg_ref) + b1_ref[...]
    row = i * B + lax.broadcasted_iota(jnp.int32, (B, 1), 0)
    hp = jnp.where(row < N, hp, 0.0)
    hpre_ref[...] = hp
    st = jnp.concatenate(
        [jnp.sum(hp, axis=0, keepdims=True), jnp.sum(hp * hp, axis=0, keepdims=True)],
        axis=0,
    )

    @pl.when(i == 0)
    def _():
        stats_ref[...] = st

    @pl.when(i > 0)
    def _():
        stats_ref[...] += st


def _tc2b_body(hpre_ref, stats_ref, deg_ref, gamma_ref, beta_ref, w2_ref, y2_ref):
    mean = stats_ref[0:1] / N
    var = stats_ref[1:2] / N - mean * mean
    h = (hpre_ref[...] - mean) * lax.rsqrt(var + BN_EPS) * gamma_ref[...] + beta_ref[...]
    h = jnp.maximum(h, 0.0)
    y2_ref[...] = (
        jnp.dot(h, w2_ref[...], preferred_element_type=jnp.float32) * _dis(deg_ref)
    )


def _tc3_body(acc_ref, y2_ref, deg_ref, b2_ref, out_ref):
    z = (acc_ref[0] + acc_ref[1] + y2_ref[...]) * _dis(deg_ref) + b2_ref[...]
    col = lax.broadcasted_iota(jnp.int32, (B, CP), 1)
    valid = col < C
    m = jnp.max(jnp.where(valid, z, -1e30), axis=1, keepdims=True)
    ex = jnp.where(valid, jnp.exp(z - m), 0.0)
    out_ref[...] = z - m - jnp.log(jnp.sum(ex, axis=1, keepdims=True))


def _deg_spec():
    return pl.BlockSpec((2, B, 16), lambda i: (0, i, 0))


def _row_spec(d):
    return pl.BlockSpec((B, d), lambda i: (i, 0))


def _full_spec(r, d):
    return pl.BlockSpec((r, d), lambda i: (0, 0))


_tc1 = pl.pallas_call(
    _tc1_body,
    grid=(G,),
    in_specs=[_deg_spec(), _row_spec(F), _full_spec(F, H)],
    out_specs=_row_spec(H),
    out_shape=jax.ShapeDtypeStruct((NP, H), jnp.float32),
)

_tc2a = pl.pallas_call(
    _tc2a_body,
    grid=(G,),
    in_specs=[
        pl.BlockSpec((2, B, H), lambda i: (0, i, 0)),
        _row_spec(H),
        _deg_spec(),
        _full_spec(1, H),
    ],
    out_specs=[_row_spec(H), _full_spec(2, H)],
    out_shape=[
        jax.ShapeDtypeStruct((NP, H), jnp.float32),
        jax.ShapeDtypeStruct((2, H), jnp.float32),
    ],
)

_tc2b = pl.pallas_call(
    _tc2b_body,
    grid=(G,),
    in_specs=[
        _row_spec(H),
        _full_spec(2, H),
        _deg_spec(),
        _full_spec(1, H),
        _full_spec(1, H),
        _full_spec(H, CP),
    ],
    out_specs=_row_spec(CP),
    out_shape=jax.ShapeDtypeStruct((NP, CP), jnp.float32),
)

_tc3 = pl.pallas_call(
    _tc3_body,
    grid=(G,),
    in_specs=[
        pl.BlockSpec((2, B, CP), lambda i: (0, i, 0)),
        _row_spec(CP),
        _deg_spec(),
        _full_spec(1, CP),
    ],
    out_specs=_row_spec(CP),
    out_shape=jax.ShapeDtypeStruct((NP, CP), jnp.float32),
)


def kernel(x, edge_index, W1, b1, gamma, beta, W2, b2):
    ei = edge_index.astype(jnp.int32)
    src = ei[0]
    dst = ei[1]
    xp = jnp.pad(x, ((0, NP - N), (0, 0)))
    degacc = _make_deg()(dst).reshape(2, NP, 16)
    y1 = _tc1(degacc, xp, W1)
    acc1 = _make_prop(H)(y1, src, dst).reshape(2, NP, H)
    hpre, stats = _tc2a(acc1, y1, degacc, b1.reshape(1, H))
    W2p = jnp.pad(W2, ((0, 0), (0, CP - C)))
    y2 = _tc2b(hpre, stats, degacc, gamma.reshape(1, H), beta.reshape(1, H), W2p)
    acc2 = _make_prop(CP)(y2, src, dst).reshape(2, NP, CP)
    z = _tc3(acc2, y2, degacc, jnp.pad(b2, (0, CP - C)).reshape(1, CP))
    return z[:N, :C]
